# baseline (speedup 1.0000x reference)
import functools

import jax
import jax.numpy as jnp
from jax.experimental import pallas as pl
from jax.experimental.pallas import tpu as pltpu

EPS = 1e-5
BIG = 1e10


def _interp():
    return jax.default_backend() == "cpu"


def _fps_kernel(xT, yT, zT, ox, oy, oz, *, npoint, n):
    x = xT[...]
    y = yT[...]
    z = zT[...]
    b = x.shape[1]
    iota = jax.lax.broadcasted_iota(jnp.int32, (n, b), 0)

    def body(i, carry):
        dists, far = carry
        mask = iota == far
        cx = jnp.sum(jnp.where(mask, x, 0.0), axis=0, keepdims=True)
        cy = jnp.sum(jnp.where(mask, y, 0.0), axis=0, keepdims=True)
        cz = jnp.sum(jnp.where(mask, z, 0.0), axis=0, keepdims=True)
        ox[pl.ds(i, 1), :] = cx
        oy[pl.ds(i, 1), :] = cy
        oz[pl.ds(i, 1), :] = cz
        d = (x - cx) ** 2 + (y - cy) ** 2 + (z - cz) ** 2
        dists = jnp.minimum(dists, d)
        dmax = jnp.max(dists, axis=0, keepdims=True)
        cand = jnp.where(dists == dmax, iota, n)
        far = jnp.min(cand, axis=0, keepdims=True)
        return dists, far

    jax.lax.fori_loop(
        0, npoint, body,
        (jnp.full((n, b), BIG, jnp.float32), jnp.zeros((1, b), jnp.int32)),
    )


def _fps_call(xyz, npoint):
    bsz, n, _ = xyz.shape
    xT = jnp.transpose(xyz[..., 0])
    yT = jnp.transpose(xyz[..., 1])
    zT = jnp.transpose(xyz[..., 2])
    outs = pl.pallas_call(
        functools.partial(_fps_kernel, npoint=npoint, n=n),
        out_shape=[jax.ShapeDtypeStruct((npoint, bsz), jnp.float32)] * 3,
        interpret=_interp(),
    )(xT, yT, zT)
    return jnp.stack([o.T for o in outs], axis=-1)


def _sa_kernel(ctr_ref, ptsT_ref, gsrc_ref, *rest, k, r2, sub, wb_count,
               hoist, batch):
    wbs = rest[:wb_count]
    out_ref = rest[wb_count]
    scr = rest[wb_count + 1] if batch else None
    t = ctr_ref.shape[0]
    n = ptsT_ref.shape[1]
    ctr = ctr_ref[...]
    cx = ctr[:, 0:1]
    cy = ctr[:, 1:2]
    cz = ctr[:, 2:3]
    px = ptsT_ref[0:1, :]
    py = ptsT_ref[1:2, :]
    pz = ptsT_ref[2:3, :]
    d2 = (cx - px) ** 2 + (cy - py) ** 2 + (cz - pz) ** 2
    work0 = jnp.where(d2 < r2, d2, BIG)
    iota = jax.lax.broadcasted_iota(jnp.int32, (t, n), 1)
    gsrc = gsrc_ref[...]
    cin = gsrc.shape[1]
    if wb_count:
        cout = wbs[wb_count - 2].shape[1]
    else:
        cout = cin

    def sel(work):
        m = jnp.min(work, axis=1, keepdims=True)
        cand = jnp.where(work == m, iota, n)
        amin = jnp.min(cand, axis=1, keepdims=True)
        return iota == amin, m < 1e9

    oh0, _ = sel(work0)
    oh0f = oh0.astype(jnp.float32)
    if hoist:
        hsrc = gsrc
        for li in range(wb_count // 2):
            hsrc = jnp.dot(hsrc, wbs[2 * li][...],
                           preferred_element_type=jnp.float32)
            hsrc = jnp.maximum(hsrc + wbs[2 * li + 1][...], 0.0)
        gsrc = hsrc
    if sub:
        if cin > 3:
            ctrpad = jnp.concatenate(
                [ctr[:, :3], jnp.zeros((t, cin - 3), jnp.float32)], axis=1)
        else:
            ctrpad = ctr[:, :cin]

    def body(i, carry):
        work, mx = carry
        ohi, valid = sel(work)
        work = jnp.where(ohi, BIG, work)
        ohf = jnp.where(valid, ohi.astype(jnp.float32), oh0f)
        g = jnp.dot(ohf, gsrc, preferred_element_type=jnp.float32)
        if sub:
            g = g - ctrpad
        if batch:
            scr[i] = g
            return work, mx
        h = g
        if not hoist:
            for li in range(wb_count // 2):
                w = wbs[2 * li][...]
                b = wbs[2 * li + 1][...]
                h = jnp.dot(h, w, preferred_element_type=jnp.float32) + b
                h = jnp.maximum(h, 0.0)
        return work, jnp.maximum(mx, h)

    _, mx = jax.lax.fori_loop(
        0, k, body, (work0, jnp.full((t, cout), -jnp.inf, jnp.float32)))
    if batch:
        h = scr[...].reshape(k * t, cin)
        for li in range(wb_count // 2):
            h = jnp.dot(h, wbs[2 * li][...],
                        preferred_element_type=jnp.float32)
            h = jnp.maximum(h + wbs[2 * li + 1][...], 0.0)
        mx = jnp.max(h.reshape(k, t, cout), axis=0)
    out_ref[...] = mx


def _fold_bn(layers):
    out = []
    for (w, b, g, be, m, v) in layers:
        s = g / jnp.sqrt(v + EPS)
        out.append((w * s[None, :], ((b - m) * s + be)[None, :]))
    return out


def _sa_call(ctr, xyz, gsrc, wbs, k, radius, sub, hoist=False):
    bsz, np_, _ = ctr.shape
    n = xyz.shape[1]
    cin = gsrc.shape[2]
    if wbs:
        cout = wbs[-1][0].shape[1]
    else:
        cout = cin
    tile = min(np_, 128)
    grid = (bsz, np_ // tile)
    ptsT = jnp.transpose(xyz, (0, 2, 1))
    flat_w = [a for wb in wbs for a in wb]
    batch = (not hoist) and bool(wbs)
    in_specs = [
        pl.BlockSpec((None, tile, 3), lambda b, t: (b, t, 0)),
        pl.BlockSpec((None, 3, n), lambda b, t: (b, 0, 0)),
        pl.BlockSpec((None, n, cin), lambda b, t: (b, 0, 0)),
    ] + [pl.BlockSpec(a.shape, lambda b, t: (0, 0)) for a in flat_w]
    return pl.pallas_call(
        functools.partial(_sa_kernel, k=k, r2=radius * radius, sub=sub,
                          wb_count=len(flat_w), hoist=hoist, batch=batch),
        grid=grid,
        in_specs=in_specs,
        out_specs=pl.BlockSpec((None, tile, cout), lambda b, t: (b, t, 0)),
        out_shape=jax.ShapeDtypeStruct((bsz, np_, cout), jnp.float32),
        scratch_shapes=([pltpu.VMEM((k, tile, cin), jnp.float32)]
                        if batch else []),
        interpret=_interp(),
    )(ctr, ptsT, gsrc, *flat_w)


def _head_kernel(x_ref, *rest, nblocks, sigmoid):
    out_ref = rest[-1]
    h = x_ref[...]
    p = 0
    for _ in range(nblocks):
        w = rest[p][...]
        b = rest[p + 1][...]
        s = rest[p + 2][...]
        tt = rest[p + 3][...]
        p += 4
        h = jnp.maximum(jnp.dot(h, w, preferred_element_type=jnp.float32) + b,
                        0.0)
        h = h * s + tt
    wf = rest[p][...]
    bf = rest[p + 1][...]
    o = jnp.dot(h, wf, preferred_element_type=jnp.float32) + bf
    if sigmoid:
        o = jax.nn.sigmoid(o)
    out_ref[...] = o


def _head_call(x, seq, sigmoid=False):
    bsz, np_, cin = x.shape
    flat = []
    for (w, b, g, be, m, v) in seq['blocks']:
        s = g / jnp.sqrt(v + EPS)
        flat += [w, b[None, :], s[:, None], (be - m * s)[:, None]]
    wf, bf = seq['final']
    flat += [wf, bf[None, :]]
    cout = wf.shape[1]
    nblocks = len(seq['blocks'])
    in_specs = [pl.BlockSpec((None, np_, cin), lambda b: (b, 0, 0))] + [
        pl.BlockSpec(a.shape, lambda b: (0, 0)) for a in flat]
    return pl.pallas_call(
        functools.partial(_head_kernel, nblocks=nblocks, sigmoid=sigmoid),
        grid=(bsz,),
        in_specs=in_specs,
        out_specs=pl.BlockSpec((None, np_, cout), lambda b: (b, 0, 0)),
        out_shape=jax.ShapeDtypeStruct((bsz, np_, cout), jnp.float32),
        interpret=_interp(),
    )(x, *flat)


_VDIMS = (38, 24, 18)


_PLANE = 26 * 40


def _voxmean_kernel(xyzT_ref, featx_ref, out_ref, out3_ref, *, vt):
    tv = pl.program_id(1)
    x = xyzT_ref[0:1, :]
    y = xyzT_ref[1:2, :]
    z = xyzT_ref[2:3, :]
    dx, dy, dz = _VDIMS

    def vidx(coord, start, dim):
        vi = jnp.floor((coord - start) * (1.0 / 0.3)).astype(jnp.int32)
        return jnp.clip(vi, 0, dim - 1)

    flat = ((vidx(z, -2.4, dz) + 1) * _PLANE + (vidx(y, -3.6, dy) + 1) * 40
            + (vidx(x, -5.6, dx) + 1))
    p = featx_ref.shape[0]
    rows = jax.lax.broadcasted_iota(jnp.int32, (vt, p), 0) + tv * vt
    oh = (rows == flat).astype(jnp.float32)
    sums = jnp.dot(oh, featx_ref[...], preferred_element_type=jnp.float32)
    c = sums.shape[1] - 1
    cnt = jnp.maximum(sums[:, c:c + 1], 1.0)
    out_ref[...] = sums[:, :c - 3] / cnt
    out3_ref[...] = sums[:, c - 3:c] / cnt


def _voxmean_call(feat_pm, xyz):
    bsz, p, c = feat_pm.shape
    v = 20 * _PLANE
    vt = 800
    xyzT = jnp.transpose(xyz, (0, 2, 1))
    featx = jnp.concatenate(
        [feat_pm, jnp.ones((bsz, p, 1), jnp.float32)], axis=2)
    return pl.pallas_call(
        functools.partial(_voxmean_kernel, vt=vt),
        grid=(bsz, v // vt),
        in_specs=[
            pl.BlockSpec((None, 3, p), lambda b, t: (b, 0, 0)),
            pl.BlockSpec((None, p, c + 1), lambda b, t: (b, 0, 0)),
        ],
        out_specs=[
            pl.BlockSpec((None, vt, c - 3), lambda b, t: (b, t, 0)),
            pl.BlockSpec((None, vt, 3), lambda b, t: (b, t, 0)),
        ],
        out_shape=[
            jax.ShapeDtypeStruct((bsz, v, c - 3), jnp.float32),
            jax.ShapeDtypeStruct((bsz, v, 3), jnp.float32),
        ],
        interpret=_interp(),
    )(xyzT, featx)


def _ring_mask(val):
    r = jax.lax.broadcasted_iota(jnp.int32, (_PLANE, 1), 0)
    y = r // 40
    x = r - y * 40
    ok = (y >= 1) & (y <= 24) & (x >= 1) & (x <= 38)
    return jnp.where(ok, val, 0.0)


_GUARD = 48


def _conv3d_kernel(a_ref, b_ref, w_ref, bias_ref, out_ref, scr, *,
                   lo, hi, relu):
    d = pl.program_id(1)
    interior = (d >= lo) & (d <= hi)

    @pl.when(interior)
    def _():
        cin = a_ref.shape[1]
        scr[0:_GUARD, :] = jnp.zeros((_GUARD, cin), jnp.float32)
        scr[_GUARD:_GUARD + 2 * _PLANE, :] = a_ref[...]
        scr[_GUARD + 2 * _PLANE:_GUARD + 4 * _PLANE, :] = b_ref[...]
        scr[_GUARD + 4 * _PLANE:, :] = jnp.zeros(
            (scr.shape[0] - _GUARD - 4 * _PLANE, cin), jnp.float32)
        cout = out_ref.shape[1]
        acc = jnp.broadcast_to(bias_ref[...], (_PLANE, cout))
        for kz in range(3):
            for ky in range(3):
                for kx in range(3):
                    s = _GUARD + kz * _PLANE + (ky - 1) * 40 + (kx - 1)
                    acc = acc + jnp.dot(scr[s:s + _PLANE, :],
                                        w_ref[kz * 9 + ky * 3 + kx],
                                        preferred_element_type=jnp.float32)
        if relu:
            acc = jnp.maximum(acc, 0.0)
        out_ref[...] = _ring_mask(acc)

    @pl.when(jnp.logical_not(interior))
    def _():
        out_ref[...] = jnp.zeros_like(out_ref)


def _conv3d_call(x, w, b, out_planes, lo, hi, relu=True):
    bsz, pin, cin = x.shape
    nblk = pin // (2 * _PLANE)
    wk = jnp.transpose(w, (2, 3, 4, 1, 0)).reshape(27, cin, w.shape[0])
    cout = w.shape[0]
    return pl.pallas_call(
        functools.partial(_conv3d_kernel, lo=lo, hi=hi, relu=relu),
        grid=(bsz, out_planes),
        in_specs=[
            pl.BlockSpec((None, 2 * _PLANE, cin),
                         lambda bb, d: (bb, jnp.clip(d - lo, 0, nblk - 1), 0)),
            pl.BlockSpec((None, 2 * _PLANE, cin),
                         lambda bb, d: (bb, jnp.clip(d - lo + 1, 0, nblk - 1), 0)),
            pl.BlockSpec((27, cin, cout), lambda bb, d: (0, 0, 0)),
            pl.BlockSpec((1, cout), lambda bb, d: (0, 0)),
        ],
        out_specs=pl.BlockSpec((None, _PLANE, cout), lambda bb, d: (bb, d, 0)),
        out_shape=jax.ShapeDtypeStruct((bsz, out_planes * _PLANE, cout),
                                       jnp.float32),
        scratch_shapes=[pltpu.VMEM((2 * _GUARD + 4 * _PLANE, cin),
                                   jnp.float32)],
        interpret=_interp(),
    )(x, x, wk, b[None, :])


def _conv3ds_kernel(a_ref, b_ref, a3_ref, b3_ref, w_ref, w3_ref, bias_ref,
                    out_ref, scr, scr3, *, lo, hi):
    d = pl.program_id(1)
    interior = (d >= lo) & (d <= hi)

    @pl.when(interior)
    def _():
        scr[0:_GUARD, :] = jnp.zeros((_GUARD, 128), jnp.float32)
        scr[_GUARD:_GUARD + 2 * _PLANE, :] = a_ref[...]
        scr[_GUARD + 2 * _PLANE:_GUARD + 4 * _PLANE, :] = b_ref[...]
        scr[_GUARD + 4 * _PLANE:, :] = jnp.zeros(
            (scr.shape[0] - _GUARD - 4 * _PLANE, 128), jnp.float32)
        scr3[0:_GUARD, :] = jnp.zeros((_GUARD, 3), jnp.float32)
        scr3[_GUARD:_GUARD + 2 * _PLANE, :] = a3_ref[...]
        scr3[_GUARD + 2 * _PLANE:_GUARD + 4 * _PLANE, :] = b3_ref[...]
        scr3[_GUARD + 4 * _PLANE:, :] = jnp.zeros(
            (scr3.shape[0] - _GUARD - 4 * _PLANE, 3), jnp.float32)
        cout = out_ref.shape[1]
        acc = jnp.broadcast_to(bias_ref[...], (_PLANE, cout))
        for kz in range(3):
            for ky in range(3):
                for kx in range(3):
                    kk = kz * 9 + ky * 3 + kx
                    s = _GUARD + kz * _PLANE + (ky - 1) * 40 + (kx - 1)
                    acc = acc + jnp.dot(scr[s:s + _PLANE, :], w_ref[kk],
                                        preferred_element_type=jnp.float32)
                    for cc in range(3):
                        acc = acc + (scr3[s:s + _PLANE, cc:cc + 1]
                                     * w3_ref[kk * 3 + cc:kk * 3 + cc + 1, :])
        out_ref[...] = _ring_mask(jnp.maximum(acc, 0.0))

    @pl.when(jnp.logical_not(interior))
    def _():
        out_ref[...] = jnp.zeros_like(out_ref)


def _conv3ds_call(x128, x3, w, b, out_planes, lo, hi):
    bsz, pin, _ = x128.shape
    nblk = pin // (2 * _PLANE)
    cout = w.shape[0]
    wk = jnp.transpose(w[:, :128], (2, 3, 4, 1, 0)).reshape(27, 128, cout)
    w3 = jnp.transpose(w[:, 128:131], (2, 3, 4, 1, 0)).reshape(81, cout)
    blk = lambda cc: pl.BlockSpec(
        (None, 2 * _PLANE, cc), lambda bb, d: (bb, jnp.clip(d - lo, 0, nblk - 1), 0))
    blkp1 = lambda cc: pl.BlockSpec(
        (None, 2 * _PLANE, cc),
        lambda bb, d: (bb, jnp.clip(d - lo + 1, 0, nblk - 1), 0))
    return pl.pallas_call(
        functools.partial(_conv3ds_kernel, lo=lo, hi=hi),
        grid=(bsz, out_planes),
        in_specs=[
            blk(128), blkp1(128), blk(3), blkp1(3),
            pl.BlockSpec((27, 128, cout), lambda bb, d: (0, 0, 0)),
            pl.BlockSpec((81, cout), lambda bb, d: (0, 0)),
            pl.BlockSpec((1, cout), lambda bb, d: (0, 0)),
        ],
        out_specs=pl.BlockSpec((None, _PLANE, cout), lambda bb, d: (bb, d, 0)),
        out_shape=jax.ShapeDtypeStruct((bsz, out_planes * _PLANE, cout),
                                       jnp.float32),
        scratch_shapes=[
            pltpu.VMEM((2 * _GUARD + 4 * _PLANE, 128), jnp.float32),
            pltpu.VMEM((2 * _GUARD + 4 * _PLANE, 3), jnp.float32),
        ],
        interpret=_interp(),
    )(x128, x128, x3, x3, wk, w3, b[None, :])


def _conv2d_kernel(x_ref, w_ref, bias_ref, out_ref, scr, *, relu, sig0):
    cin = x_ref.shape[1]
    scr[0:_GUARD, :] = jnp.zeros((_GUARD, cin), jnp.float32)
    scr[_GUARD:_GUARD + _PLANE, :] = x_ref[...]
    scr[_GUARD + _PLANE:, :] = jnp.zeros(
        (scr.shape[0] - _GUARD - _PLANE, cin), jnp.float32)
    cout = out_ref.shape[1]
    acc = jnp.broadcast_to(bias_ref[...], (_PLANE, cout))
    for ky in range(3):
        for kx in range(3):
            s = _GUARD + (ky - 1) * 40 + (kx - 1)
            acc = acc + jnp.dot(scr[s:s + _PLANE, :], w_ref[ky * 3 + kx],
                                preferred_element_type=jnp.float32)
    if relu:
        acc = jnp.maximum(acc, 0.0)
    if sig0:
        ci = jax.lax.broadcasted_iota(jnp.int32, acc.shape, 1)
        acc = jnp.where(ci == 0, jax.nn.sigmoid(acc), acc)
    out_ref[...] = _ring_mask(acc)


def _conv2d_call(x, w, b, relu=True, sig0=False):
    bsz, _, cin = x.shape
    cout = w.shape[0]
    wk = jnp.transpose(w, (2, 3, 1, 0)).reshape(9, cin, cout)
    return pl.pallas_call(
        functools.partial(_conv2d_kernel, relu=relu, sig0=sig0),
        grid=(bsz,),
        in_specs=[
            pl.BlockSpec((None, _PLANE, cin), lambda bb: (bb, 0, 0)),
            pl.BlockSpec((9, cin, cout), lambda bb: (0, 0, 0)),
            pl.BlockSpec((1, cout), lambda bb: (0, 0)),
        ],
        out_specs=pl.BlockSpec((None, _PLANE, cout), lambda bb: (bb, 0, 0)),
        out_shape=jax.ShapeDtypeStruct((bsz, _PLANE, cout), jnp.float32),
        scratch_shapes=[pltpu.VMEM((2 * _GUARD + _PLANE, cin), jnp.float32)],
        interpret=_interp(),
    )(x, wk, b[None, :])


def kernel(template, search, params):
    def backbone(pc, npoints, mlps):
        xyz0 = pc[..., :3]
        c0 = _fps_call(xyz0, npoints[0])
        f0 = _sa_call(c0, xyz0, xyz0, _fold_bn(mlps[0]), 32, 0.3, True)
        c1 = _fps_call(c0, npoints[1])
        f1 = _sa_call(c1, c0, f0, _fold_bn(mlps[1]), 32, 0.5, False,
                      hoist=True)
        c2 = c1[:, :npoints[2]]
        f2 = _sa_call(c2, c1, f1, _fold_bn(mlps[2]), 32, 0.7, False,
                      hoist=True)
        return c2, f2

    nt = template.shape[1]
    ns = search.shape[1]
    mlps = [params['sa0'], params['sa1'], params['sa2']]
    t_xyz, t_feat = backbone(template, [nt // 2, nt // 4, nt // 8], mlps)
    s_xyz, s_feat = backbone(search, [ns // 2, ns // 4, ns // 8], mlps)

    fus = s_feat
    search_xyz = s_xyz

    score_pm = _head_call(fus, params['fc_cla'], sigmoid=True)
    fxf = jnp.concatenate([search_xyz, fus], axis=2)
    off = _head_call(fxf, params['vote'])
    offset = off[:, :, :3]
    fus = fus + off[:, :, 3:]
    temp_sel = search_xyz - offset

    tpool = _sa_call(temp_sel, t_xyz,
                     jnp.concatenate([t_xyz, t_feat], axis=2), [], 8, 1.0, True)
    spool = _sa_call(search_xyz, s_xyz,
                     jnp.concatenate([s_xyz, s_feat], axis=2), [], 8, 1.0, True)

    pf = jnp.concatenate([score_pm, tpool, spool, fus], axis=2)
    po = _head_call(pf, params['proposal'])
    po = jnp.concatenate([po, search_xyz], axis=2)

    bsz = po.shape[0]
    vox128, vox3 = _voxmean_call(po, search_xyz)
    cml = params['cml']
    x = _conv3ds_call(vox128, vox3, cml[0][0], cml[0][1], 12, 1, 9)
    x = _conv3d_call(x, cml[1][0], cml[1][1], 8, 1, 5)
    x = _conv3d_call(x, cml[2][0], cml[2][1], 3, 0, 2)
    xs = jnp.concatenate(
        [x[:, 0:_PLANE], x[:, _PLANE:2 * _PLANE], x[:, 2 * _PLANE:3 * _PLANE]],
        axis=2)
    wst, bst = params['rpn']['stem']
    perm = jnp.array([(j % 64) * 3 + j // 64 for j in range(192)])
    hh = _conv2d_call(xs, wst[:, perm], bst)
    wh = jnp.concatenate([params['rpn']['hm'][0], params['rpn']['loc'][0],
                          params['rpn']['z'][0]], axis=0)
    bh = jnp.concatenate([params['rpn']['hm'][1], params['rpn']['loc'][1],
                          params['rpn']['z'][1]], axis=0)
    oh = _conv2d_call(hh, wh, bh, relu=False, sig0=True)
    o = oh.reshape(bsz, 26, 40, 5)[:, 1:25, 1:39, :]
    o = jnp.transpose(o, (0, 3, 1, 2))
    return o[:, 0:1], o[:, 1:4], o[:, 4:5]


# speedup vs baseline: 1.3110x; 1.3110x over previous
import functools

import jax
import jax.numpy as jnp
from jax.experimental import pallas as pl
from jax.experimental.pallas import tpu as pltpu

EPS = 1e-5
BIG = 1e10


def _interp():
    return jax.default_backend() == "cpu"


def _fps_kernel(xT, yT, zT, ox, oy, oz, *, npoint, n):
    x = xT[...]
    y = yT[...]
    z = zT[...]
    b = x.shape[1]
    iota = jax.lax.broadcasted_iota(jnp.int32, (n, b), 0)

    def body(i, carry):
        dists, far = carry
        mask = iota == far
        cx = jnp.sum(jnp.where(mask, x, 0.0), axis=0, keepdims=True)
        cy = jnp.sum(jnp.where(mask, y, 0.0), axis=0, keepdims=True)
        cz = jnp.sum(jnp.where(mask, z, 0.0), axis=0, keepdims=True)
        ox[pl.ds(i, 1), :] = cx
        oy[pl.ds(i, 1), :] = cy
        oz[pl.ds(i, 1), :] = cz
        d = (x - cx) ** 2 + (y - cy) ** 2 + (z - cz) ** 2
        dists = jnp.minimum(dists, d)
        dmax = jnp.max(dists, axis=0, keepdims=True)
        cand = jnp.where(dists == dmax, iota, n)
        far = jnp.min(cand, axis=0, keepdims=True)
        return dists, far

    jax.lax.fori_loop(
        0, npoint, body,
        (jnp.full((n, b), BIG, jnp.float32), jnp.zeros((1, b), jnp.int32)),
    )


def _fps_call(xyz, npoint):
    bsz, n, _ = xyz.shape
    xT = jnp.transpose(xyz[..., 0])
    yT = jnp.transpose(xyz[..., 1])
    zT = jnp.transpose(xyz[..., 2])
    outs = pl.pallas_call(
        functools.partial(_fps_kernel, npoint=npoint, n=n),
        out_shape=[jax.ShapeDtypeStruct((npoint, bsz), jnp.float32)] * 3,
        interpret=_interp(),
    )(xT, yT, zT)
    return jnp.stack([o.T for o in outs], axis=-1)


def _sa_kernel(ctr_ref, ptsT_ref, gsrc_ref, *rest, k, r2, sub, wb_count,
               hoist, batch):
    wbs = rest[:wb_count]
    out_ref = rest[wb_count]
    scr = rest[wb_count + 1] if batch else None
    t = ctr_ref.shape[0]
    n = ptsT_ref.shape[1]
    ctr = ctr_ref[...]
    cx = ctr[:, 0:1]
    cy = ctr[:, 1:2]
    cz = ctr[:, 2:3]
    px = ptsT_ref[0:1, :]
    py = ptsT_ref[1:2, :]
    pz = ptsT_ref[2:3, :]
    d2 = (cx - px) ** 2 + (cy - py) ** 2 + (cz - pz) ** 2
    work0 = jnp.where(d2 < r2, d2, BIG)
    iota = jax.lax.broadcasted_iota(jnp.int32, (t, n), 1)
    gsrc = gsrc_ref[...]
    cin = gsrc.shape[1]
    if wb_count:
        cout = wbs[wb_count - 2].shape[1]
    else:
        cout = cin

    def sel(work):
        m = jnp.min(work, axis=1, keepdims=True)
        cand = jnp.where(work == m, iota, n)
        amin = jnp.min(cand, axis=1, keepdims=True)
        return iota == amin, m < 1e9

    oh0, _ = sel(work0)
    oh0f = oh0.astype(jnp.float32)
    if hoist:
        hsrc = gsrc
        for li in range(wb_count // 2):
            hsrc = jnp.dot(hsrc, wbs[2 * li][...],
                           preferred_element_type=jnp.float32)
            hsrc = jnp.maximum(hsrc + wbs[2 * li + 1][...], 0.0)
        gsrc = hsrc
    if sub:
        if cin > 3:
            ctrpad = jnp.concatenate(
                [ctr[:, :3], jnp.zeros((t, cin - 3), jnp.float32)], axis=1)
        else:
            ctrpad = ctr[:, :cin]

    def body(i, carry):
        work, mx = carry
        ohi, valid = sel(work)
        work = jnp.where(ohi, BIG, work)
        ohf = jnp.where(valid, ohi.astype(jnp.float32), oh0f)
        g = jnp.dot(ohf, gsrc, preferred_element_type=jnp.float32)
        if sub:
            g = g - ctrpad
        if batch:
            scr[i] = g
            return work, mx
        h = g
        if not hoist:
            for li in range(wb_count // 2):
                w = wbs[2 * li][...]
                b = wbs[2 * li + 1][...]
                h = jnp.dot(h, w, preferred_element_type=jnp.float32) + b
                h = jnp.maximum(h, 0.0)
        return work, jnp.maximum(mx, h)

    _, mx = jax.lax.fori_loop(
        0, k, body, (work0, jnp.full((t, cout), -jnp.inf, jnp.float32)))
    if batch:
        h = scr[...].reshape(k * t, cin)
        for li in range(wb_count // 2):
            h = jnp.dot(h, wbs[2 * li][...],
                        preferred_element_type=jnp.float32)
            h = jnp.maximum(h + wbs[2 * li + 1][...], 0.0)
        mx = jnp.max(h.reshape(k, t, cout), axis=0)
    out_ref[...] = mx


def _fold_bn(layers):
    out = []
    for (w, b, g, be, m, v) in layers:
        s = g / jnp.sqrt(v + EPS)
        out.append((w * s[None, :], ((b - m) * s + be)[None, :]))
    return out


def _sa_call(ctr, xyz, gsrc, wbs, k, radius, sub, hoist=False):
    bsz, np_, _ = ctr.shape
    n = xyz.shape[1]
    cin = gsrc.shape[2]
    if wbs:
        cout = wbs[-1][0].shape[1]
    else:
        cout = cin
    tile = min(np_, 128)
    grid = (bsz, np_ // tile)
    ptsT = jnp.transpose(xyz, (0, 2, 1))
    flat_w = [a for wb in wbs for a in wb]
    batch = (not hoist) and bool(wbs)
    in_specs = [
        pl.BlockSpec((None, tile, 3), lambda b, t: (b, t, 0)),
        pl.BlockSpec((None, 3, n), lambda b, t: (b, 0, 0)),
        pl.BlockSpec((None, n, cin), lambda b, t: (b, 0, 0)),
    ] + [pl.BlockSpec(a.shape, lambda b, t: (0, 0)) for a in flat_w]
    return pl.pallas_call(
        functools.partial(_sa_kernel, k=k, r2=radius * radius, sub=sub,
                          wb_count=len(flat_w), hoist=hoist, batch=batch),
        grid=grid,
        in_specs=in_specs,
        out_specs=pl.BlockSpec((None, tile, cout), lambda b, t: (b, t, 0)),
        out_shape=jax.ShapeDtypeStruct((bsz, np_, cout), jnp.float32),
        scratch_shapes=([pltpu.VMEM((k, tile, cin), jnp.float32)]
                        if batch else []),
        interpret=_interp(),
    )(ctr, ptsT, gsrc, *flat_w)


def _head_kernel(x_ref, *rest, nblocks, sigmoid):
    out_ref = rest[-1]
    h = x_ref[...]
    p = 0
    for _ in range(nblocks):
        w = rest[p][...]
        b = rest[p + 1][...]
        s = rest[p + 2][...]
        tt = rest[p + 3][...]
        p += 4
        h = jnp.maximum(jnp.dot(h, w, preferred_element_type=jnp.float32) + b,
                        0.0)
        h = h * s + tt
    wf = rest[p][...]
    bf = rest[p + 1][...]
    o = jnp.dot(h, wf, preferred_element_type=jnp.float32) + bf
    if sigmoid:
        o = jax.nn.sigmoid(o)
    out_ref[...] = o


def _head_call(x, seq, sigmoid=False):
    bsz, np_, cin = x.shape
    flat = []
    for (w, b, g, be, m, v) in seq['blocks']:
        s = g / jnp.sqrt(v + EPS)
        flat += [w, b[None, :], s[:, None], (be - m * s)[:, None]]
    wf, bf = seq['final']
    flat += [wf, bf[None, :]]
    cout = wf.shape[1]
    nblocks = len(seq['blocks'])
    in_specs = [pl.BlockSpec((None, np_, cin), lambda b: (b, 0, 0))] + [
        pl.BlockSpec(a.shape, lambda b: (0, 0)) for a in flat]
    return pl.pallas_call(
        functools.partial(_head_kernel, nblocks=nblocks, sigmoid=sigmoid),
        grid=(bsz,),
        in_specs=in_specs,
        out_specs=pl.BlockSpec((None, np_, cout), lambda b: (b, 0, 0)),
        out_shape=jax.ShapeDtypeStruct((bsz, np_, cout), jnp.float32),
        interpret=_interp(),
    )(x, *flat)


_VDIMS = (38, 24, 18)


_PLANE = 26 * 40


def _voxmean_kernel(xyzT_ref, featx_ref, out_ref, *, vt):
    tv = pl.program_id(1)
    x = xyzT_ref[0:1, :]
    y = xyzT_ref[1:2, :]
    z = xyzT_ref[2:3, :]
    dx, dy, dz = _VDIMS

    def vidx(coord, start, dim):
        vi = jnp.floor((coord - start) * (1.0 / 0.3)).astype(jnp.int32)
        return jnp.clip(vi, 0, dim - 1)

    flat = ((vidx(z, -2.4, dz) + 1) * _PLANE + (vidx(y, -3.6, dy) + 1) * 40
            + (vidx(x, -5.6, dx) + 1))
    p = featx_ref.shape[0]
    rows = jax.lax.broadcasted_iota(jnp.int32, (vt, p), 0) + tv * vt
    oh = (rows == flat).astype(jnp.float32)
    sums = jnp.dot(oh, featx_ref[...], preferred_element_type=jnp.float32)
    c = sums.shape[1] - 1
    cnt = jnp.maximum(sums[:, c:c + 1], 1.0)
    out_ref[...] = sums[:, :c] / cnt


def _voxmean_call(feat_pm, xyz):
    bsz, p, c = feat_pm.shape
    v = 20 * _PLANE
    vt = 800
    xyzT = jnp.transpose(xyz, (0, 2, 1))
    featx = jnp.concatenate(
        [feat_pm, jnp.ones((bsz, p, 1), jnp.float32)], axis=2)
    return pl.pallas_call(
        functools.partial(_voxmean_kernel, vt=vt),
        grid=(bsz, v // vt),
        in_specs=[
            pl.BlockSpec((None, 3, p), lambda b, t: (b, 0, 0)),
            pl.BlockSpec((None, p, c + 1), lambda b, t: (b, 0, 0)),
        ],
        out_specs=pl.BlockSpec((None, vt, c), lambda b, t: (b, t, 0)),
        out_shape=jax.ShapeDtypeStruct((bsz, v, c), jnp.float32),
        interpret=_interp(),
    )(xyzT, featx)


def _ring_mask(val):
    r = jax.lax.broadcasted_iota(jnp.int32, (_PLANE, 1), 0)
    y = r // 40
    x = r - y * 40
    ok = (y >= 1) & (y <= 24) & (x >= 1) & (x <= 38)
    return jnp.where(ok, val, 0.0)


_GUARD = 48


def _conv3d_kernel(a_ref, b_ref, w_ref, bias_ref, out_ref, scr, *,
                   lo, hi, relu):
    d = pl.program_id(1)
    interior = (d >= lo) & (d <= hi)

    @pl.when(interior)
    def _():
        cin = a_ref.shape[1]
        scr[0:_GUARD, :] = jnp.zeros((_GUARD, cin), jnp.float32)
        scr[_GUARD:_GUARD + 2 * _PLANE, :] = a_ref[...]
        scr[_GUARD + 2 * _PLANE:_GUARD + 4 * _PLANE, :] = b_ref[...]
        scr[_GUARD + 4 * _PLANE:, :] = jnp.zeros(
            (scr.shape[0] - _GUARD - 4 * _PLANE, cin), jnp.float32)
        cout = out_ref.shape[1]
        acc = jnp.broadcast_to(bias_ref[...], (_PLANE, cout))
        for kz in range(3):
            for ky in range(3):
                for kx in range(3):
                    s = _GUARD + kz * _PLANE + (ky - 1) * 40 + (kx - 1)
                    acc = acc + jnp.dot(scr[s:s + _PLANE, :],
                                        w_ref[kz * 9 + ky * 3 + kx],
                                        preferred_element_type=jnp.float32)
        if relu:
            acc = jnp.maximum(acc, 0.0)
        out_ref[...] = _ring_mask(acc)

    @pl.when(jnp.logical_not(interior))
    def _():
        out_ref[...] = jnp.zeros_like(out_ref)


def _conv3d_call(x, w, b, out_planes, lo, hi, relu=True):
    bsz, pin, cin = x.shape
    nblk = pin // (2 * _PLANE)
    wk = jnp.transpose(w, (2, 3, 4, 1, 0)).reshape(27, cin, w.shape[0])
    cout = w.shape[0]
    return pl.pallas_call(
        functools.partial(_conv3d_kernel, lo=lo, hi=hi, relu=relu),
        grid=(bsz, out_planes),
        in_specs=[
            pl.BlockSpec((None, 2 * _PLANE, cin),
                         lambda bb, d: (bb, jnp.clip(d - lo, 0, nblk - 1), 0)),
            pl.BlockSpec((None, 2 * _PLANE, cin),
                         lambda bb, d: (bb, jnp.clip(d - lo + 1, 0, nblk - 1), 0)),
            pl.BlockSpec((27, cin, cout), lambda bb, d: (0, 0, 0)),
            pl.BlockSpec((1, cout), lambda bb, d: (0, 0)),
        ],
        out_specs=pl.BlockSpec((None, _PLANE, cout), lambda bb, d: (bb, d, 0)),
        out_shape=jax.ShapeDtypeStruct((bsz, out_planes * _PLANE, cout),
                                       jnp.float32),
        scratch_shapes=[pltpu.VMEM((2 * _GUARD + 4 * _PLANE, cin),
                                   jnp.float32)],
        interpret=_interp(),
    )(x, x, wk, b[None, :])


def _conv2d_kernel(x_ref, w_ref, bias_ref, out_ref, scr, *, relu, sig0):
    cin = x_ref.shape[1]
    scr[0:_GUARD, :] = jnp.zeros((_GUARD, cin), jnp.float32)
    scr[_GUARD:_GUARD + _PLANE, :] = x_ref[...]
    scr[_GUARD + _PLANE:, :] = jnp.zeros(
        (scr.shape[0] - _GUARD - _PLANE, cin), jnp.float32)
    cout = out_ref.shape[1]
    acc = jnp.broadcast_to(bias_ref[...], (_PLANE, cout))
    for ky in range(3):
        for kx in range(3):
            s = _GUARD + (ky - 1) * 40 + (kx - 1)
            acc = acc + jnp.dot(scr[s:s + _PLANE, :], w_ref[ky * 3 + kx],
                                preferred_element_type=jnp.float32)
    if relu:
        acc = jnp.maximum(acc, 0.0)
    if sig0:
        ci = jax.lax.broadcasted_iota(jnp.int32, acc.shape, 1)
        acc = jnp.where(ci == 0, jax.nn.sigmoid(acc), acc)
    out_ref[...] = _ring_mask(acc)


def _conv2d_call(x, w, b, relu=True, sig0=False):
    bsz, _, cin = x.shape
    cout = w.shape[0]
    wk = jnp.transpose(w, (2, 3, 1, 0)).reshape(9, cin, cout)
    return pl.pallas_call(
        functools.partial(_conv2d_kernel, relu=relu, sig0=sig0),
        grid=(bsz,),
        in_specs=[
            pl.BlockSpec((None, _PLANE, cin), lambda bb: (bb, 0, 0)),
            pl.BlockSpec((9, cin, cout), lambda bb: (0, 0, 0)),
            pl.BlockSpec((1, cout), lambda bb: (0, 0)),
        ],
        out_specs=pl.BlockSpec((None, _PLANE, cout), lambda bb: (bb, 0, 0)),
        out_shape=jax.ShapeDtypeStruct((bsz, _PLANE, cout), jnp.float32),
        scratch_shapes=[pltpu.VMEM((2 * _GUARD + _PLANE, cin), jnp.float32)],
        interpret=_interp(),
    )(x, wk, b[None, :])


def kernel(template, search, params):
    def backbone(pc, npoints, mlps):
        xyz0 = pc[..., :3]
        c0 = _fps_call(xyz0, npoints[0])
        f0 = _sa_call(c0, xyz0, xyz0, _fold_bn(mlps[0]), 32, 0.3, True)
        c1 = _fps_call(c0, npoints[1])
        f1 = _sa_call(c1, c0, f0, _fold_bn(mlps[1]), 32, 0.5, False,
                      hoist=True)
        c2 = c1[:, :npoints[2]]
        f2 = _sa_call(c2, c1, f1, _fold_bn(mlps[2]), 32, 0.7, False,
                      hoist=True)
        return c2, f2

    nt = template.shape[1]
    ns = search.shape[1]
    mlps = [params['sa0'], params['sa1'], params['sa2']]
    t_xyz, t_feat = backbone(template, [nt // 2, nt // 4, nt // 8], mlps)
    s_xyz, s_feat = backbone(search, [ns // 2, ns // 4, ns // 8], mlps)

    fus = s_feat
    search_xyz = s_xyz

    score_pm = _head_call(fus, params['fc_cla'], sigmoid=True)
    fxf = jnp.concatenate([search_xyz, fus], axis=2)
    off = _head_call(fxf, params['vote'])
    offset = off[:, :, :3]
    fus = fus + off[:, :, 3:]
    temp_sel = search_xyz - offset

    tpool = _sa_call(temp_sel, t_xyz,
                     jnp.concatenate([t_xyz, t_feat], axis=2), [], 8, 1.0, True)
    spool = _sa_call(search_xyz, s_xyz,
                     jnp.concatenate([s_xyz, s_feat], axis=2), [], 8, 1.0, True)

    pf = jnp.concatenate([score_pm, tpool, spool, fus], axis=2)
    po = _head_call(pf, params['proposal'])
    po = jnp.concatenate([po, search_xyz], axis=2)

    bsz = po.shape[0]
    vox = _voxmean_call(po, search_xyz)
    cml = params['cml']
    x = _conv3d_call(vox, cml[0][0], cml[0][1], 12, 1, 9)
    x = _conv3d_call(x, cml[1][0], cml[1][1], 8, 1, 5)
    x = _conv3d_call(x, cml[2][0], cml[2][1], 3, 0, 2)
    xs = jnp.concatenate(
        [x[:, 0:_PLANE], x[:, _PLANE:2 * _PLANE], x[:, 2 * _PLANE:3 * _PLANE]],
        axis=2)
    wst, bst = params['rpn']['stem']
    perm = jnp.array([(j % 64) * 3 + j // 64 for j in range(192)])
    hh = _conv2d_call(xs, wst[:, perm], bst)
    wh = jnp.concatenate([params['rpn']['hm'][0], params['rpn']['loc'][0],
                          params['rpn']['z'][0]], axis=0)
    bh = jnp.concatenate([params['rpn']['hm'][1], params['rpn']['loc'][1],
                          params['rpn']['z'][1]], axis=0)
    oh = _conv2d_call(hh, wh, bh, relu=False, sig0=True)
    o = oh.reshape(bsz, 26, 40, 5)[:, 1:25, 1:39, :]
    o = jnp.transpose(o, (0, 3, 1, 2))
    return o[:, 0:1], o[:, 1:4], o[:, 4:5]


# speedup vs baseline: 1.3120x; 1.0008x over previous
import functools

import jax
import jax.numpy as jnp
from jax.experimental import pallas as pl
from jax.experimental.pallas import tpu as pltpu

EPS = 1e-5
BIG = 1e10


def _fps_kernel(xT, yT, zT, ox, oy, oz, *, npoint, n):
    x = xT[...]
    y = yT[...]
    z = zT[...]
    b = x.shape[1]
    iota = jax.lax.broadcasted_iota(jnp.int32, (n, b), 0)

    def body(i, carry):
        dists, far = carry
        mask = iota == far
        cx = jnp.sum(jnp.where(mask, x, 0.0), axis=0, keepdims=True)
        cy = jnp.sum(jnp.where(mask, y, 0.0), axis=0, keepdims=True)
        cz = jnp.sum(jnp.where(mask, z, 0.0), axis=0, keepdims=True)
        ox[pl.ds(i, 1), :] = cx
        oy[pl.ds(i, 1), :] = cy
        oz[pl.ds(i, 1), :] = cz
        d = (x - cx) ** 2 + (y - cy) ** 2 + (z - cz) ** 2
        dists = jnp.minimum(dists, d)
        dmax = jnp.max(dists, axis=0, keepdims=True)
        cand = jnp.where(dists == dmax, iota, n)
        far = jnp.min(cand, axis=0, keepdims=True)
        return dists, far

    jax.lax.fori_loop(
        0, npoint, body,
        (jnp.full((n, b), BIG, jnp.float32), jnp.zeros((1, b), jnp.int32)),
    )


def _fps_call(xyz, npoint):
    bsz, n, _ = xyz.shape
    xT = jnp.transpose(xyz[..., 0])
    yT = jnp.transpose(xyz[..., 1])
    zT = jnp.transpose(xyz[..., 2])
    outs = pl.pallas_call(
        functools.partial(_fps_kernel, npoint=npoint, n=n),
        out_shape=[jax.ShapeDtypeStruct((npoint, bsz), jnp.float32)] * 3,
    )(xT, yT, zT)
    return jnp.stack([o.T for o in outs], axis=-1)


def _sa_kernel(ctr_ref, ptsT_ref, gsrc_ref, *rest, k, r2, sub, wb_count,
               hoist, batch):
    wbs = rest[:wb_count]
    out_ref = rest[wb_count]
    scr = rest[wb_count + 1] if batch else None
    t = ctr_ref.shape[0]
    n = ptsT_ref.shape[1]
    ctr = ctr_ref[...]
    cx = ctr[:, 0:1]
    cy = ctr[:, 1:2]
    cz = ctr[:, 2:3]
    px = ptsT_ref[0:1, :]
    py = ptsT_ref[1:2, :]
    pz = ptsT_ref[2:3, :]
    d2 = (cx - px) ** 2 + (cy - py) ** 2 + (cz - pz) ** 2
    work0 = jnp.where(d2 < r2, d2, BIG)
    iota = jax.lax.broadcasted_iota(jnp.int32, (t, n), 1)
    gsrc = gsrc_ref[...]
    cin = gsrc.shape[1]
    if wb_count:
        cout = wbs[wb_count - 2].shape[1]
    else:
        cout = cin

    def sel(work):
        m = jnp.min(work, axis=1, keepdims=True)
        cand = jnp.where(work == m, iota, n)
        amin = jnp.min(cand, axis=1, keepdims=True)
        return iota == amin, m < 1e9

    oh0, _ = sel(work0)
    oh0f = oh0.astype(jnp.float32)
    if hoist:
        hsrc = gsrc
        for li in range(wb_count // 2):
            hsrc = jnp.dot(hsrc, wbs[2 * li][...],
                           preferred_element_type=jnp.float32)
            hsrc = jnp.maximum(hsrc + wbs[2 * li + 1][...], 0.0)
        gsrc = hsrc
    if sub:
        if cin > 3:
            ctrpad = jnp.concatenate(
                [ctr[:, :3], jnp.zeros((t, cin - 3), jnp.float32)], axis=1)
        else:
            ctrpad = ctr[:, :cin]

    def body(i, carry):
        work, mx = carry
        ohi, valid = sel(work)
        work = jnp.where(ohi, BIG, work)
        ohf = jnp.where(valid, ohi.astype(jnp.float32), oh0f)
        g = jnp.dot(ohf, gsrc, preferred_element_type=jnp.float32)
        if sub:
            g = g - ctrpad
        if batch:
            scr[i] = g
            return work, mx
        h = g
        if not hoist:
            for li in range(wb_count // 2):
                w = wbs[2 * li][...]
                b = wbs[2 * li + 1][...]
                h = jnp.dot(h, w, preferred_element_type=jnp.float32) + b
                h = jnp.maximum(h, 0.0)
        return work, jnp.maximum(mx, h)

    _, mx = jax.lax.fori_loop(
        0, k, body, (work0, jnp.full((t, cout), -jnp.inf, jnp.float32)))
    if batch:
        h = scr[...].reshape(k * t, cin)
        for li in range(wb_count // 2):
            h = jnp.dot(h, wbs[2 * li][...],
                        preferred_element_type=jnp.float32)
            h = jnp.maximum(h + wbs[2 * li + 1][...], 0.0)
        mx = jnp.max(h.reshape(k, t, cout), axis=0)
    out_ref[...] = mx


def _fold_bn(layers):
    out = []
    for (w, b, g, be, m, v) in layers:
        s = g / jnp.sqrt(v + EPS)
        out.append((w * s[None, :], ((b - m) * s + be)[None, :]))
    return out


def _sa_call(ctr, xyz, gsrc, wbs, k, radius, sub, hoist=False):
    bsz, np_, _ = ctr.shape
    n = xyz.shape[1]
    cin = gsrc.shape[2]
    if wbs:
        cout = wbs[-1][0].shape[1]
    else:
        cout = cin
    tile = min(np_, 128)
    grid = (bsz, np_ // tile)
    ptsT = jnp.transpose(xyz, (0, 2, 1))
    flat_w = [a for wb in wbs for a in wb]
    batch = (not hoist) and bool(wbs)
    in_specs = [
        pl.BlockSpec((None, tile, 3), lambda b, t: (b, t, 0)),
        pl.BlockSpec((None, 3, n), lambda b, t: (b, 0, 0)),
        pl.BlockSpec((None, n, cin), lambda b, t: (b, 0, 0)),
    ] + [pl.BlockSpec(a.shape, lambda b, t: (0, 0)) for a in flat_w]
    return pl.pallas_call(
        functools.partial(_sa_kernel, k=k, r2=radius * radius, sub=sub,
                          wb_count=len(flat_w), hoist=hoist, batch=batch),
        grid=grid,
        in_specs=in_specs,
        out_specs=pl.BlockSpec((None, tile, cout), lambda b, t: (b, t, 0)),
        out_shape=jax.ShapeDtypeStruct((bsz, np_, cout), jnp.float32),
        scratch_shapes=([pltpu.VMEM((k, tile, cin), jnp.float32)]
                        if batch else []),
    )(ctr, ptsT, gsrc, *flat_w)


def _head_kernel(x_ref, *rest, nblocks, sigmoid):
    out_ref = rest[-1]
    h = x_ref[...]
    p = 0
    for _ in range(nblocks):
        w = rest[p][...]
        b = rest[p + 1][...]
        s = rest[p + 2][...]
        tt = rest[p + 3][...]
        p += 4
        h = jnp.maximum(jnp.dot(h, w, preferred_element_type=jnp.float32) + b,
                        0.0)
        h = h * s + tt
    wf = rest[p][...]
    bf = rest[p + 1][...]
    o = jnp.dot(h, wf, preferred_element_type=jnp.float32) + bf
    if sigmoid:
        o = jax.nn.sigmoid(o)
    out_ref[...] = o


def _head_call(x, seq, sigmoid=False):
    bsz, np_, cin = x.shape
    flat = []
    for (w, b, g, be, m, v) in seq['blocks']:
        s = g / jnp.sqrt(v + EPS)
        flat += [w, b[None, :], s[:, None], (be - m * s)[:, None]]
    wf, bf = seq['final']
    flat += [wf, bf[None, :]]
    cout = wf.shape[1]
    nblocks = len(seq['blocks'])
    in_specs = [pl.BlockSpec((None, np_, cin), lambda b: (b, 0, 0))] + [
        pl.BlockSpec(a.shape, lambda b: (0, 0)) for a in flat]
    return pl.pallas_call(
        functools.partial(_head_kernel, nblocks=nblocks, sigmoid=sigmoid),
        grid=(bsz,),
        in_specs=in_specs,
        out_specs=pl.BlockSpec((None, np_, cout), lambda b: (b, 0, 0)),
        out_shape=jax.ShapeDtypeStruct((bsz, np_, cout), jnp.float32),
    )(x, *flat)


_VDIMS = (38, 24, 18)


_PLANE = 26 * 40


def _voxmean_kernel(xyzT_ref, featx_ref, out_ref, *, vt):
    tv = pl.program_id(1)
    x = xyzT_ref[0:1, :]
    y = xyzT_ref[1:2, :]
    z = xyzT_ref[2:3, :]
    dx, dy, dz = _VDIMS

    def vidx(coord, start, dim):
        vi = jnp.floor((coord - start) * (1.0 / 0.3)).astype(jnp.int32)
        return jnp.clip(vi, 0, dim - 1)

    flat = ((vidx(z, -2.4, dz) + 1) * _PLANE + (vidx(y, -3.6, dy) + 1) * 40
            + (vidx(x, -5.6, dx) + 1))
    p = featx_ref.shape[0]
    rows = jax.lax.broadcasted_iota(jnp.int32, (vt, p), 0) + tv * vt
    oh = (rows == flat).astype(jnp.float32)
    sums = jnp.dot(oh, featx_ref[...], preferred_element_type=jnp.float32)
    c = sums.shape[1] - 1
    cnt = jnp.maximum(sums[:, c:c + 1], 1.0)
    out_ref[...] = sums[:, :c] / cnt


def _voxmean_call(feat_pm, xyz):
    bsz, p, c = feat_pm.shape
    v = 20 * _PLANE
    vt = 800
    xyzT = jnp.transpose(xyz, (0, 2, 1))
    featx = jnp.concatenate(
        [feat_pm, jnp.ones((bsz, p, 1), jnp.float32)], axis=2)
    return pl.pallas_call(
        functools.partial(_voxmean_kernel, vt=vt),
        grid=(bsz, v // vt),
        in_specs=[
            pl.BlockSpec((None, 3, p), lambda b, t: (b, 0, 0)),
            pl.BlockSpec((None, p, c + 1), lambda b, t: (b, 0, 0)),
        ],
        out_specs=pl.BlockSpec((None, vt, c), lambda b, t: (b, t, 0)),
        out_shape=jax.ShapeDtypeStruct((bsz, v, c), jnp.float32),
    )(xyzT, featx)


def _ring_mask(val):
    r = jax.lax.broadcasted_iota(jnp.int32, (_PLANE, 1), 0)
    y = r // 40
    x = r - y * 40
    ok = (y >= 1) & (y <= 24) & (x >= 1) & (x <= 38)
    return jnp.where(ok, val, 0.0)


_GUARD = 48


def _conv3d_kernel(a_ref, b_ref, w_ref, bias_ref, out_ref, scr, *,
                   lo, hi, relu):
    d = pl.program_id(1)
    interior = (d >= lo) & (d <= hi)

    @pl.when(interior)
    def _():
        cin = a_ref.shape[1]
        scr[0:_GUARD, :] = jnp.zeros((_GUARD, cin), jnp.float32)
        scr[_GUARD:_GUARD + 2 * _PLANE, :] = a_ref[...]
        scr[_GUARD + 2 * _PLANE:_GUARD + 4 * _PLANE, :] = b_ref[...]
        scr[_GUARD + 4 * _PLANE:, :] = jnp.zeros(
            (scr.shape[0] - _GUARD - 4 * _PLANE, cin), jnp.float32)
        cout = out_ref.shape[1]
        acc = jnp.broadcast_to(bias_ref[...], (_PLANE, cout))
        for kz in range(3):
            for ky in range(3):
                for kx in range(3):
                    s = _GUARD + kz * _PLANE + (ky - 1) * 40 + (kx - 1)
                    acc = acc + jnp.dot(scr[s:s + _PLANE, :],
                                        w_ref[kz * 9 + ky * 3 + kx],
                                        preferred_element_type=jnp.float32)
        if relu:
            acc = jnp.maximum(acc, 0.0)
        out_ref[...] = _ring_mask(acc)

    @pl.when(jnp.logical_not(interior))
    def _():
        out_ref[...] = jnp.zeros_like(out_ref)


def _conv3d_call(x, w, b, out_planes, lo, hi, relu=True):
    bsz, pin, cin = x.shape
    nblk = pin // (2 * _PLANE)
    wk = jnp.transpose(w, (2, 3, 4, 1, 0)).reshape(27, cin, w.shape[0])
    cout = w.shape[0]
    return pl.pallas_call(
        functools.partial(_conv3d_kernel, lo=lo, hi=hi, relu=relu),
        grid=(bsz, out_planes),
        in_specs=[
            pl.BlockSpec((None, 2 * _PLANE, cin),
                         lambda bb, d: (bb, jnp.clip(d - lo, 0, nblk - 1), 0)),
            pl.BlockSpec((None, 2 * _PLANE, cin),
                         lambda bb, d: (bb, jnp.clip(d - lo + 1, 0, nblk - 1), 0)),
            pl.BlockSpec((27, cin, cout), lambda bb, d: (0, 0, 0)),
            pl.BlockSpec((1, cout), lambda bb, d: (0, 0)),
        ],
        out_specs=pl.BlockSpec((None, _PLANE, cout), lambda bb, d: (bb, d, 0)),
        out_shape=jax.ShapeDtypeStruct((bsz, out_planes * _PLANE, cout),
                                       jnp.float32),
        scratch_shapes=[pltpu.VMEM((2 * _GUARD + 4 * _PLANE, cin),
                                   jnp.float32)],
    )(x, x, wk, b[None, :])


def _conv2d_kernel(x_ref, w_ref, bias_ref, out_ref, scr, *, relu, sig0):
    cin = x_ref.shape[1]
    scr[0:_GUARD, :] = jnp.zeros((_GUARD, cin), jnp.float32)
    scr[_GUARD:_GUARD + _PLANE, :] = x_ref[...]
    scr[_GUARD + _PLANE:, :] = jnp.zeros(
        (scr.shape[0] - _GUARD - _PLANE, cin), jnp.float32)
    cout = out_ref.shape[1]
    acc = jnp.broadcast_to(bias_ref[...], (_PLANE, cout))
    for ky in range(3):
        for kx in range(3):
            s = _GUARD + (ky - 1) * 40 + (kx - 1)
            acc = acc + jnp.dot(scr[s:s + _PLANE, :], w_ref[ky * 3 + kx],
                                preferred_element_type=jnp.float32)
    if relu:
        acc = jnp.maximum(acc, 0.0)
    if sig0:
        ci = jax.lax.broadcasted_iota(jnp.int32, acc.shape, 1)
        acc = jnp.where(ci == 0, jax.nn.sigmoid(acc), acc)
    out_ref[...] = _ring_mask(acc)


def _conv2d_call(x, w, b, relu=True, sig0=False):
    bsz, _, cin = x.shape
    cout = w.shape[0]
    wk = jnp.transpose(w, (2, 3, 1, 0)).reshape(9, cin, cout)
    return pl.pallas_call(
        functools.partial(_conv2d_kernel, relu=relu, sig0=sig0),
        grid=(bsz,),
        in_specs=[
            pl.BlockSpec((None, _PLANE, cin), lambda bb: (bb, 0, 0)),
            pl.BlockSpec((9, cin, cout), lambda bb: (0, 0, 0)),
            pl.BlockSpec((1, cout), lambda bb: (0, 0)),
        ],
        out_specs=pl.BlockSpec((None, _PLANE, cout), lambda bb: (bb, 0, 0)),
        out_shape=jax.ShapeDtypeStruct((bsz, _PLANE, cout), jnp.float32),
        scratch_shapes=[pltpu.VMEM((2 * _GUARD + _PLANE, cin), jnp.float32)],
    )(x, wk, b[None, :])


def kernel(template, search, params):
    def backbone(pc, npoints, mlps):
        xyz0 = pc[..., :3]
        c0 = _fps_call(xyz0, npoints[0])
        f0 = _sa_call(c0, xyz0, xyz0, _fold_bn(mlps[0]), 32, 0.3, True)
        c1 = _fps_call(c0, npoints[1])
        f1 = _sa_call(c1, c0, f0, _fold_bn(mlps[1]), 32, 0.5, False,
                      hoist=True)
        c2 = c1[:, :npoints[2]]
        f2 = _sa_call(c2, c1, f1, _fold_bn(mlps[2]), 32, 0.7, False,
                      hoist=True)
        return c2, f2

    nt = template.shape[1]
    ns = search.shape[1]
    mlps = [params['sa0'], params['sa1'], params['sa2']]
    t_xyz, t_feat = backbone(template, [nt // 2, nt // 4, nt // 8], mlps)
    s_xyz, s_feat = backbone(search, [ns // 2, ns // 4, ns // 8], mlps)

    fus = s_feat
    search_xyz = s_xyz

    score_pm = _head_call(fus, params['fc_cla'], sigmoid=True)
    fxf = jnp.concatenate([search_xyz, fus], axis=2)
    off = _head_call(fxf, params['vote'])
    offset = off[:, :, :3]
    fus = fus + off[:, :, 3:]
    temp_sel = search_xyz - offset

    tpool = _sa_call(temp_sel, t_xyz,
                     jnp.concatenate([t_xyz, t_feat], axis=2), [], 8, 1.0, True)
    spool = _sa_call(search_xyz, s_xyz,
                     jnp.concatenate([s_xyz, s_feat], axis=2), [], 8, 1.0, True)

    pf = jnp.concatenate([score_pm, tpool, spool, fus], axis=2)
    po = _head_call(pf, params['proposal'])
    po = jnp.concatenate([po, search_xyz], axis=2)

    bsz = po.shape[0]
    vox = _voxmean_call(po, search_xyz)
    cml = params['cml']
    x = _conv3d_call(vox, cml[0][0], cml[0][1], 12, 1, 9)
    x = _conv3d_call(x, cml[1][0], cml[1][1], 8, 1, 5)
    x = _conv3d_call(x, cml[2][0], cml[2][1], 3, 0, 2)
    xs = jnp.concatenate(
        [x[:, 0:_PLANE], x[:, _PLANE:2 * _PLANE], x[:, 2 * _PLANE:3 * _PLANE]],
        axis=2)
    wst, bst = params['rpn']['stem']
    perm = jnp.array([(j % 64) * 3 + j // 64 for j in range(192)])
    hh = _conv2d_call(xs, wst[:, perm], bst)
    wh = jnp.concatenate([params['rpn']['hm'][0], params['rpn']['loc'][0],
                          params['rpn']['z'][0]], axis=0)
    bh = jnp.concatenate([params['rpn']['hm'][1], params['rpn']['loc'][1],
                          params['rpn']['z'][1]], axis=0)
    oh = _conv2d_call(hh, wh, bh, relu=False, sig0=True)
    o = oh.reshape(bsz, 26, 40, 5)[:, 1:25, 1:39, :]
    o = jnp.transpose(o, (0, 3, 1, 2))
    return o[:, 0:1], o[:, 1:4], o[:, 4:5]
